# baseline (speedup 1.0000x reference)
import functools

import jax
import jax.numpy as jnp
from jax import lax
from jax.experimental import pallas as pl
from jax.experimental.pallas import tpu as pltpu
from jax.experimental.pallas import tpu_sc as plsc

D = 64
B = 16384
NF = 5
CH = 128

_info = plsc.get_sparse_core_info()
NC = _info.num_cores
NS = _info.num_subcores
NW = NC * NS
BPW = B // NW
NCH = BPW // CH
T = NF * NCH

_mesh = plsc.VectorSubcoreMesh(core_axis_name="c", subcore_axis_name="s")


@functools.partial(
    pl.kernel,
    out_type=jax.ShapeDtypeStruct((B, NF, D), jnp.float32),
    mesh=_mesh,
    scratch_types=[
        pltpu.VMEM((NF, NCH, CH), jnp.int32),
        pltpu.VMEM((CH, 1, D), jnp.float32),
        pltpu.VMEM((CH, 1, D), jnp.float32),
        pltpu.SemaphoreType.DMA,
        pltpu.SemaphoreType.DMA,
        pltpu.SemaphoreType.DMA,
        pltpu.SemaphoreType.DMA,
    ],
)
def _gather_concat(idx_hbm, Wu, Wg, Wa, Wo, Wz, out_hbm,
                   idx_v, rows0, rows1, sg0, sg1, sw0, sw1):
    tables = (Wu, Wg, Wa, Wo, Wz)
    rows = (rows0, rows1)
    gsems = (sg0, sg1)
    wsems = (sw0, sw1)

    wid = lax.axis_index("s") * NC + lax.axis_index("c")
    row0 = wid * BPW

    for f in range(NF):
        pltpu.sync_copy(idx_hbm.at[f, pl.ds(wid * NCH, NCH)], idx_v.at[f])

    def gstart(t):
        f, j = divmod(t, NCH)
        return pltpu.async_copy(
            tables[f].at[idx_v.at[f, j]], rows[t % 2], gsems[t % 2])

    def wstart(t):
        f, j = divmod(t, NCH)
        return pltpu.async_copy(
            rows[t % 2], out_hbm.at[pl.ds(row0 + j * CH, CH), pl.ds(f, 1)],
            wsems[t % 2])

    gcs = [None] * T
    wcs = [None] * T
    gcs[0] = gstart(0)
    for t in range(T):
        if t + 1 < T:
            if t - 1 >= 0:
                wcs[t - 1].wait()
            gcs[t + 1] = gstart(t + 1)
        gcs[t].wait()
        wcs[t] = wstart(t)
    wcs[T - 2].wait()
    wcs[T - 1].wait()


def kernel(user_id, gender, age, occupation, zip_code,
           W_user_id, W_gender, W_age, W_occupation, W_zip_code):
    idx = jnp.stack([user_id, gender, age, occupation, zip_code])
    idx = idx.reshape(NF, B // CH, CH)
    tables3 = [w.reshape(w.shape[0], 1, D)
               for w in (W_user_id, W_gender, W_age, W_occupation, W_zip_code)]
    out = _gather_concat(idx, *tables3)
    return out.reshape(B, NF * D)
